# baseline (speedup 1.0000x reference)
import functools

import jax
import jax.numpy as jnp
from jax import lax
from jax.experimental import pallas as pl
from jax.experimental.pallas import tpu as pltpu
from jax.experimental.pallas import tpu_sc as plsc

BATCH = 4096
SEQ_LEN = 200
EMBED_DIM = 128
VOCAB = 4097
TOTAL = BATCH * SEQ_LEN

NUM_CORES = 2
NUM_SUBCORES = 16
NUM_WORKERS = NUM_CORES * NUM_SUBCORES
PER_WORKER = TOTAL // NUM_WORKERS

CHUNK = 80
NBUF = 8
GAHEAD = 2
SWAIT = NBUF - GAHEAD
NUM_CHUNKS = PER_WORKER // CHUNK
NUM_GROUPS = NUM_CHUNKS // NBUF


def _embed_kernel(table_hbm, seq_hbm, out_hbm, table_sh, idx_v, rows_v,
                  isem, *sems):
    gsem = sems[:NBUF]
    ssem = sems[NBUF:]
    cid = lax.axis_index("c")
    sid = lax.axis_index("s")
    wid = sid * NUM_CORES + cid
    base = wid * PER_WORKER

    def icopy_desc(i, b):
        return pltpu.make_async_copy(
            seq_hbm.at[pl.ds(base + i * CHUNK, CHUNK)],
            idx_v.at[pl.ds(b * CHUNK, CHUNK)],
            isem,
        )

    def gather_desc(i, b):
        return pltpu.make_async_copy(
            table_sh.at[idx_v.at[pl.ds(b * CHUNK, CHUNK)]],
            rows_v.at[pl.ds(b * CHUNK, CHUNK)],
            gsem[b],
        )

    def store_desc(i, b):
        return pltpu.make_async_copy(
            rows_v.at[pl.ds(b * CHUNK, CHUNK)],
            out_hbm.at[pl.ds(base + i * CHUNK, CHUNK)],
            ssem[b],
        )

    for b in range(NBUF):
        icopy_desc(b, b).start()
    pltpu.sync_copy(table_hbm.at[pl.ds(sid * 256, 256)],
                    table_sh.at[pl.ds(sid * 256, 256)])

    @pl.when(sid == NUM_SUBCORES - 1)
    def _():
        pltpu.sync_copy(table_hbm.at[pl.ds(VOCAB - 1, 1)],
                        table_sh.at[pl.ds(VOCAB - 1, 1)])

    plsc.subcore_barrier()

    for j in range(GAHEAD):
        icopy_desc(j, j).wait()
        gather_desc(j, j).start()

    def step(i, b, has_swait=True, has_gahead=True, has_icopy=True):
        sg = (b + GAHEAD) % NBUF
        gather_desc(i, b).wait()
        store_desc(i, b).start()
        if has_swait:
            store_desc(i - SWAIT, sg).wait()
        if has_gahead:
            icopy_desc(i + GAHEAD, sg).wait()
            gather_desc(i + GAHEAD, sg).start()
        if has_icopy:
            icopy_desc(i + NBUF, b).start()

    for b in range(NBUF):
        step(b, b, has_swait=(b >= SWAIT))

    def group_body(g, carry):
        i0 = g * NBUF
        for b in range(NBUF):
            step(i0 + b, b)
        return carry

    lax.fori_loop(1, NUM_GROUPS - 1, group_body, 0)

    i0 = (NUM_GROUPS - 1) * NBUF
    for b in range(NBUF):
        step(i0 + b, b,
             has_gahead=(b + GAHEAD < NBUF),
             has_icopy=False)
    for j in range(NUM_CHUNKS - SWAIT, NUM_CHUNKS):
        store_desc(j, j % NBUF).wait()


@jax.jit
def _embed(sequences_flat, table):
    mesh = plsc.VectorSubcoreMesh(core_axis_name="c", subcore_axis_name="s")
    k = functools.partial(
        pl.kernel,
        mesh=mesh,
        out_type=jax.ShapeDtypeStruct((TOTAL, EMBED_DIM), jnp.float32),
        scratch_types=[
            pltpu.VMEM_SHARED((VOCAB, EMBED_DIM), jnp.float32),
            pltpu.VMEM((NBUF * CHUNK,), jnp.int32),
            pltpu.VMEM((NBUF * CHUNK, EMBED_DIM), jnp.float32),
        ] + [pltpu.SemaphoreType.DMA] * (1 + 2 * NBUF),
    )(_embed_kernel)
    return k(table, sequences_flat)


def kernel(sequences, table):
    flat = sequences.reshape(TOTAL).astype(jnp.int32)
    out = _embed(flat, table)
    return out.reshape(BATCH, SEQ_LEN, EMBED_DIM)


# speedup vs baseline: 1.0092x; 1.0092x over previous
import functools

import jax
import jax.numpy as jnp
from jax import lax
from jax.experimental import pallas as pl
from jax.experimental.pallas import tpu as pltpu
from jax.experimental.pallas import tpu_sc as plsc

BATCH = 4096
SEQ_LEN = 200
EMBED_DIM = 128
VOCAB = 4097
TOTAL = BATCH * SEQ_LEN

NUM_CORES = 2
NUM_SUBCORES = 16
NUM_WORKERS = NUM_CORES * NUM_SUBCORES
PER_WORKER = TOTAL // NUM_WORKERS

CHUNK = 64
NBUF = 10
GAHEAD = 5
SWAIT = NBUF - GAHEAD
NUM_CHUNKS = PER_WORKER // CHUNK
NUM_GROUPS = NUM_CHUNKS // NBUF


def _embed_kernel(table_hbm, seq_hbm, out_hbm, table_sh, idx_v, rows_v,
                  isem, *sems):
    gsem = sems[:NBUF]
    ssem = sems[NBUF:]
    cid = lax.axis_index("c")
    sid = lax.axis_index("s")
    wid = sid * NUM_CORES + cid
    base = wid * PER_WORKER

    def icopy_desc(i, b):
        return pltpu.make_async_copy(
            seq_hbm.at[pl.ds(base + i * CHUNK, CHUNK)],
            idx_v.at[pl.ds(b * CHUNK, CHUNK)],
            isem,
        )

    def gather_desc(i, b):
        return pltpu.make_async_copy(
            table_sh.at[idx_v.at[pl.ds(b * CHUNK, CHUNK)]],
            rows_v.at[pl.ds(b * CHUNK, CHUNK)],
            gsem[b],
        )

    def store_desc(i, b):
        return pltpu.make_async_copy(
            rows_v.at[pl.ds(b * CHUNK, CHUNK)],
            out_hbm.at[pl.ds(base + i * CHUNK, CHUNK)],
            ssem[b],
        )

    for b in range(NBUF):
        icopy_desc(b, b).start()
    pltpu.sync_copy(table_hbm.at[pl.ds(sid * 256, 256)],
                    table_sh.at[pl.ds(sid * 256, 256)])

    @pl.when(sid == NUM_SUBCORES - 1)
    def _():
        pltpu.sync_copy(table_hbm.at[pl.ds(VOCAB - 1, 1)],
                        table_sh.at[pl.ds(VOCAB - 1, 1)])

    plsc.subcore_barrier()

    for j in range(GAHEAD):
        icopy_desc(j, j).wait()
        gather_desc(j, j).start()

    def step(i, b, has_swait=True, has_gahead=True, has_icopy=True):
        sg = (b + GAHEAD) % NBUF
        gather_desc(i, b).wait()
        store_desc(i, b).start()
        if has_swait:
            store_desc(i - SWAIT, sg).wait()
        if has_gahead:
            icopy_desc(i + GAHEAD, sg).wait()
            gather_desc(i + GAHEAD, sg).start()
        if has_icopy:
            icopy_desc(i + NBUF, b).start()

    for b in range(NBUF):
        step(b, b, has_swait=(b >= SWAIT))

    def group_body(g, carry):
        i0 = g * NBUF
        for b in range(NBUF):
            step(i0 + b, b)
        return carry

    lax.fori_loop(1, NUM_GROUPS - 1, group_body, 0)

    i0 = (NUM_GROUPS - 1) * NBUF
    for b in range(NBUF):
        step(i0 + b, b,
             has_gahead=(b + GAHEAD < NBUF),
             has_icopy=False)
    for j in range(NUM_CHUNKS - SWAIT, NUM_CHUNKS):
        store_desc(j, j % NBUF).wait()


@jax.jit
def _embed(sequences_flat, table):
    mesh = plsc.VectorSubcoreMesh(core_axis_name="c", subcore_axis_name="s")
    k = functools.partial(
        pl.kernel,
        mesh=mesh,
        out_type=jax.ShapeDtypeStruct((TOTAL, EMBED_DIM), jnp.float32),
        scratch_types=[
            pltpu.VMEM_SHARED((VOCAB, EMBED_DIM), jnp.float32),
            pltpu.VMEM((NBUF * CHUNK,), jnp.int32),
            pltpu.VMEM((NBUF * CHUNK, EMBED_DIM), jnp.float32),
        ] + [pltpu.SemaphoreType.DMA] * (1 + 2 * NBUF),
    )(_embed_kernel)
    return k(table, sequences_flat)


def kernel(sequences, table):
    flat = sequences.reshape(TOTAL).astype(jnp.int32)
    out = _embed(flat, table)
    return out.reshape(BATCH, SEQ_LEN, EMBED_DIM)
